# Initial kernel scaffold; baseline (speedup 1.0000x reference)
import jax
import jax.numpy as jnp
from jax.experimental import pallas as pl


def kernel(x_user, x_item, edge_index_user_rates_item, edge_index_item_rated_by_user, Wl1_ui, Wr1_ui, b1_ui, Wl2_ui, Wr2_ui, b2_ui, Wl1_iu, Wr1_iu, b1_iu, Wl2_iu, Wr2_iu, b2_iu, g1_user, be1_user, g2_user, be2_user, Rw_user, Rb_user, Pw_user, Pb_user, g1_item, be1_item, g2_item, be2_item, Rw_item, Rb_item, Pw_item, Pb_item):
    raise NotImplementedError("write your pallas kernel here")


# speedup vs baseline: 3.1201x; 3.1201x over previous
import functools

import jax
import jax.numpy as jnp
from jax import lax
from jax.experimental import pallas as pl
from jax.experimental.pallas import tpu as pltpu
from jax.experimental.pallas import tpu_sc as plsc

N = 10000
E = 320000
D = 128
H = 256
O = 128

NP = 10240
CH = 128
NTILE = 16
NCORE = 2
IB = 32
NCHUNK = 160
EW = NCHUNK * CH
EPAD = NTILE * EW
RPT = NP // NTILE


def _prep_edges(ei_ui, ei_iu):
    def prep(ei, off):
        pad = EPAD - E
        src = jnp.concatenate([ei[0] + off, jnp.zeros((pad,), jnp.int32)])
        dst = jnp.concatenate([ei[1], jnp.full((pad,), N, jnp.int32)])
        return src.reshape(NTILE, NCHUNK, CH), dst.reshape(NTILE, NCHUNK, CH)

    su, du = prep(ei_ui, 0)
    si, di = prep(ei_iu, N)
    return jnp.stack([su, si]), jnp.stack([du, di])


def _sc_segsum(table, srcs, dsts, with_counts):
    mesh = plsc.VectorSubcoreMesh(core_axis_name="c", subcore_axis_name="s")
    out_type = [jax.ShapeDtypeStruct((NCORE, NP, D), jnp.float32)]
    if with_counts:
        out_type.append(jax.ShapeDtypeStruct((NCORE, NP), jnp.float32))
    scratch = [
        pltpu.VMEM((IB, CH), jnp.int32),
        pltpu.VMEM((IB, CH), jnp.int32),
        pltpu.VMEM((CH, D), jnp.float32),
        pltpu.VMEM((CH,), jnp.float32),
        pltpu.VMEM_SHARED((NP, D), jnp.float32),
        pltpu.VMEM_SHARED((NP,), jnp.float32),
        pltpu.SemaphoreType.DMA,
    ]

    @functools.partial(pl.kernel, out_type=out_type, mesh=mesh,
                       scratch_types=scratch)
    def k(table_h, srcs_h, dsts_h, zrows_h, zcnt_h, *rest):
        if with_counts:
            sums_h, counts_h, src_v, dst_v, rows_v, ones_v, acc, cacc, sem = rest
        else:
            sums_h, src_v, dst_v, rows_v, ones_v, acc, cacc, sem = rest
        cid = lax.axis_index("c")
        sid = lax.axis_index("s")
        sl = pl.ds(sid * RPT, RPT)

        pltpu.sync_copy(zrows_h.at[sl], acc.at[sl])
        if with_counts:
            pltpu.sync_copy(zcnt_h.at[sl], cacc.at[sl])
            for i in range(CH // 16):
                ones_v[pl.ds(i * 16, 16)] = jnp.ones((16,), jnp.float32)

        plsc.subcore_barrier()

        def outer(i, carry):
            pltpu.sync_copy(srcs_h.at[cid, sid, pl.ds(i * IB, IB)], src_v)
            pltpu.sync_copy(dsts_h.at[cid, sid, pl.ds(i * IB, IB)], dst_v)

            def body(j, carry2):
                pltpu.async_copy(table_h.at[src_v.at[j]], rows_v, sem).wait()
                pltpu.sync_copy(rows_v, acc.at[dst_v.at[j]], add=True)
                if with_counts:
                    pltpu.sync_copy(ones_v, cacc.at[dst_v.at[j]], add=True)
                return carry2

            return lax.fori_loop(0, IB, body, carry)

        lax.fori_loop(0, NCHUNK // IB, outer, 0)
        plsc.subcore_barrier()

        pltpu.sync_copy(acc.at[sl], sums_h.at[cid, sl])
        if with_counts:
            pltpu.sync_copy(cacc.at[sl], counts_h.at[cid, sl])

    zrows = jnp.zeros((NP, D), jnp.float32)
    zcnt = jnp.zeros((NP,), jnp.float32)
    out = k(table, srcs, dsts, zrows, zcnt)
    return tuple(jax.tree.leaves(out))


_BR = 1280


def _tc1_body(s_ref, c_ref, x_ref, wl1_ref, wr1_ref, b1_ref, g1_ref, be1_ref,
              wl2_ref, wc_ref, bc_ref, y_ref, w_ref):
    cnt = jnp.maximum(c_ref[0], 1.0)
    agg = s_ref[0] / cnt
    h = (jnp.dot(agg, wl1_ref[0], preferred_element_type=jnp.float32)
         + jnp.dot(x_ref[0], wr1_ref[0], preferred_element_type=jnp.float32)
         + b1_ref[0])
    m = jnp.mean(h, axis=-1, keepdims=True)
    v = jnp.mean((h - m) ** 2, axis=-1, keepdims=True)
    x1 = jnp.maximum((h - m) * lax.rsqrt(v + 1e-5) * g1_ref[0] + be1_ref[0], 0.0)
    y_ref[0] = jnp.dot(x1, wl2_ref[0], preferred_element_type=jnp.float32)
    w_ref[0] = (jnp.dot(x1, wc_ref[0], preferred_element_type=jnp.float32)
                + bc_ref[0])


def _tc2_body(s_ref, c_ref, w_ref, g2_ref, be2_ref, pw_ref, pb_ref, o_ref):
    cnt = jnp.maximum(c_ref[0], 1.0)
    t = s_ref[0] / cnt + w_ref[0]
    m = jnp.mean(t, axis=-1, keepdims=True)
    v = jnp.mean((t - m) ** 2, axis=-1, keepdims=True)
    x2 = (t - m) * lax.rsqrt(v + 1e-5) * g2_ref[0] + be2_ref[0]
    o_ref[0] = jnp.dot(x2, pw_ref[0], preferred_element_type=jnp.float32) + pb_ref[0]


def _rows(shape):
    return pl.BlockSpec((1, _BR) + shape[2:], lambda t, i: (t, i) + (0,) * (len(shape) - 2))


def _whole(shape):
    return pl.BlockSpec((1,) + shape[1:], lambda t, i: (t,) + (0,) * (len(shape) - 1))


def _tc_call(body, ins, n_out, out_dim):
    grid = (NCORE, NP // _BR)
    in_specs = [_rows(a.shape) if a.shape[1] == NP else _whole(a.shape) for a in ins]
    out_shape = [jax.ShapeDtypeStruct((NCORE, NP, out_dim), jnp.float32)] * n_out
    out_specs = [_rows((NCORE, NP, out_dim))] * n_out
    res = pl.pallas_call(
        body, grid=grid, in_specs=in_specs, out_specs=out_specs,
        out_shape=out_shape)(*ins)
    return res


def kernel(x_user, x_item, edge_index_user_rates_item, edge_index_item_rated_by_user,
           Wl1_ui, Wr1_ui, b1_ui, Wl2_ui, Wr2_ui, b2_ui,
           Wl1_iu, Wr1_iu, b1_iu, Wl2_iu, Wr2_iu, b2_iu,
           g1_user, be1_user, g2_user, be2_user, Rw_user, Rb_user, Pw_user, Pb_user,
           g1_item, be1_item, g2_item, be2_item, Rw_item, Rb_item, Pw_item, Pb_item):
    srcs, dsts = _prep_edges(edge_index_user_rates_item, edge_index_item_rated_by_user)
    table1 = jnp.concatenate([x_user, x_item], 0)

    pad = lambda a: jnp.pad(a, ((0, NP - N), (0, 0)))
    x = jnp.stack([pad(x_item), pad(x_user)])
    wl1 = jnp.stack([Wl1_ui, Wl1_iu])
    wr1 = jnp.stack([Wr1_ui, Wr1_iu])
    b1 = jnp.stack([b1_ui, b1_iu])[:, None, :]
    g1 = jnp.stack([g1_item, g1_user])[:, None, :]
    be1 = jnp.stack([be1_item, be1_user])[:, None, :]
    wl2 = jnp.stack([Wl2_iu, Wl2_ui])
    wc = jnp.stack([Wr2_ui + Rw_item, Wr2_iu + Rw_user])
    bc = jnp.stack([b2_ui + Rb_item, b2_iu + Rb_user])[:, None, :]
    g2 = jnp.stack([g2_item, g2_user])[:, None, :]
    be2 = jnp.stack([be2_item, be2_user])[:, None, :]
    pw = jnp.stack([Pw_item, Pw_user])
    pb = jnp.stack([Pb_item, Pb_user])[:, None, :]

    s1, counts = _sc_segsum(table1, srcs, dsts, with_counts=True)
    c = counts[:, :, None]

    y, w = _tc_call(_tc1_body, [s1, c, x, wl1, wr1, b1, g1, be1, wl2, wc, bc],
                    n_out=2, out_dim=O)

    table2 = jnp.concatenate([y[1, :N], y[0, :N]], 0)
    (s2,) = _sc_segsum(table2, srcs, dsts, with_counts=False)

    (out,) = _tc_call(_tc2_body, [s2, c, w, g2, be2, pw, pb], n_out=1, out_dim=O)
    return (out[1, :N], out[0, :N])


# speedup vs baseline: 3.3602x; 1.0770x over previous
import functools

import jax
import jax.numpy as jnp
from jax import lax
from jax.experimental import pallas as pl
from jax.experimental.pallas import tpu as pltpu
from jax.experimental.pallas import tpu_sc as plsc

N = 10000
E = 320000
D = 128
H = 256
O = 128

NP = 10240
CH = 128
NTILE = 16
NCORE = 2
NCHUNK = 160
EW = NCHUNK * CH
EPAD = NTILE * EW
RPT = NP // NTILE


def _prep_edges(ei_ui, ei_iu):
    def prep(ei, off):
        pad = EPAD - E
        src = jnp.concatenate([ei[0] + off, jnp.zeros((pad,), jnp.int32)])
        dst = jnp.concatenate([ei[1], jnp.full((pad,), N, jnp.int32)])
        return src.reshape(NTILE, NCHUNK, CH), dst.reshape(NTILE, NCHUNK, CH)

    su, du = prep(ei_ui, 0)
    si, di = prep(ei_iu, N)
    return jnp.stack([su, si]), jnp.stack([du, di])


def _sc_segsum(table, srcs, dsts, with_counts):
    mesh = plsc.VectorSubcoreMesh(core_axis_name="c", subcore_axis_name="s")
    out_type = [jax.ShapeDtypeStruct((NCORE, NP, D), jnp.float32)]
    if with_counts:
        out_type.append(jax.ShapeDtypeStruct((NCORE, NP), jnp.float32))
    scratch = [
        pltpu.VMEM((3, CH), jnp.int32),
        pltpu.VMEM((3, CH), jnp.int32),
        pltpu.VMEM((2, CH, D), jnp.float32),
        pltpu.VMEM((CH,), jnp.float32),
        pltpu.VMEM_SHARED((NP, D), jnp.float32),
        pltpu.VMEM_SHARED((NP,), jnp.float32),
        pltpu.SemaphoreType.DMA,
        pltpu.SemaphoreType.DMA,
        pltpu.SemaphoreType.DMA,
        pltpu.SemaphoreType.DMA,
    ]

    @functools.partial(pl.kernel, out_type=out_type, mesh=mesh,
                       scratch_types=scratch)
    def k(table_h, srcs_h, dsts_h, zrows_h, zcnt_h, *rest):
        if with_counts:
            (sums_h, counts_h, src_v, dst_v, rows_v, ones_v, acc, cacc,
             gsem, isem, ssem, csem) = rest
        else:
            (sums_h, src_v, dst_v, rows_v, ones_v, acc, cacc,
             gsem, isem, ssem, csem) = rest
        cid = lax.axis_index("c")
        sid = lax.axis_index("s")
        sl = pl.ds(sid * RPT, RPT)

        def stage_idx(chunk, buf, sync):
            cp = pltpu.sync_copy if sync else (
                lambda s, d: pltpu.async_copy(s, d, isem))
            cp(srcs_h.at[cid, sid, chunk], src_v.at[buf])
            cp(dsts_h.at[cid, sid, chunk], dst_v.at[buf])

        def wait_idx():
            pltpu.make_async_copy(srcs_h.at[cid, sid, 0], src_v.at[0], isem).wait()
            pltpu.make_async_copy(dsts_h.at[cid, sid, 0], dst_v.at[0], isem).wait()

        def gather(chunk_buf, row_buf):
            pltpu.async_copy(table_h.at[src_v.at[chunk_buf]],
                             rows_v.at[row_buf], gsem)

        def wait_gather():
            pltpu.make_async_copy(table_h.at[src_v.at[0]], rows_v.at[0],
                                  gsem).wait()

        def scatter(row_buf, chunk_buf):
            pltpu.async_copy(rows_v.at[row_buf], acc.at[dst_v.at[chunk_buf]],
                             ssem, add=True)
            if with_counts:
                pltpu.async_copy(ones_v, cacc.at[dst_v.at[chunk_buf]],
                                 csem, add=True)

        def wait_scatter():
            pltpu.make_async_copy(rows_v.at[0], acc.at[dst_v.at[0]], ssem).wait()
            if with_counts:
                pltpu.make_async_copy(ones_v, cacc.at[dst_v.at[0]], csem).wait()

        pltpu.sync_copy(zrows_h.at[sl], acc.at[sl])
        if with_counts:
            pltpu.sync_copy(zcnt_h.at[sl], cacc.at[sl])
            for i in range(CH // 16):
                ones_v[pl.ds(i * 16, 16)] = jnp.ones((16,), jnp.float32)

        plsc.subcore_barrier()

        stage_idx(0, 0, sync=True)
        stage_idx(1, 1, sync=False)
        gather(0, 0)

        def body(j, carry):
            @pl.when(j > 0)
            def _():
                wait_scatter()
            wait_idx()
            stage_idx((j + 2) % NCHUNK, (j + 2) % 3, sync=False)
            wait_gather()
            @pl.when(j + 1 < NCHUNK)
            def _():
                gather((j + 1) % 3, (j + 1) % 2)
            scatter(j % 2, j % 3)
            return carry

        lax.fori_loop(0, NCHUNK, body, 0)
        wait_scatter()
        wait_idx()
        plsc.subcore_barrier()

        pltpu.sync_copy(acc.at[sl], sums_h.at[cid, sl])
        if with_counts:
            pltpu.sync_copy(cacc.at[sl], counts_h.at[cid, sl])

    zrows = jnp.zeros((NP, D), jnp.float32)
    zcnt = jnp.zeros((NP,), jnp.float32)
    out = k(table, srcs, dsts, zrows, zcnt)
    return tuple(jax.tree.leaves(out))


_BR = 1280


def _tc1_body(s_ref, c_ref, x_ref, wl1_ref, wr1_ref, b1_ref, g1_ref, be1_ref,
              wl2_ref, wc_ref, bc_ref, y_ref, w_ref):
    cnt = jnp.maximum(c_ref[0], 1.0)
    agg = s_ref[0] / cnt
    h = (jnp.dot(agg, wl1_ref[0], preferred_element_type=jnp.float32)
         + jnp.dot(x_ref[0], wr1_ref[0], preferred_element_type=jnp.float32)
         + b1_ref[0])
    m = jnp.mean(h, axis=-1, keepdims=True)
    v = jnp.mean((h - m) ** 2, axis=-1, keepdims=True)
    x1 = jnp.maximum((h - m) * lax.rsqrt(v + 1e-5) * g1_ref[0] + be1_ref[0], 0.0)
    y_ref[0] = jnp.dot(x1, wl2_ref[0], preferred_element_type=jnp.float32)
    w_ref[0] = (jnp.dot(x1, wc_ref[0], preferred_element_type=jnp.float32)
                + bc_ref[0])


def _tc2_body(s_ref, c_ref, w_ref, g2_ref, be2_ref, pw_ref, pb_ref, o_ref):
    cnt = jnp.maximum(c_ref[0], 1.0)
    t = s_ref[0] / cnt + w_ref[0]
    m = jnp.mean(t, axis=-1, keepdims=True)
    v = jnp.mean((t - m) ** 2, axis=-1, keepdims=True)
    x2 = (t - m) * lax.rsqrt(v + 1e-5) * g2_ref[0] + be2_ref[0]
    o_ref[0] = jnp.dot(x2, pw_ref[0], preferred_element_type=jnp.float32) + pb_ref[0]


def _rows(shape):
    return pl.BlockSpec((1, _BR) + shape[2:], lambda t, i: (t, i) + (0,) * (len(shape) - 2))


def _whole(shape):
    return pl.BlockSpec((1,) + shape[1:], lambda t, i: (t,) + (0,) * (len(shape) - 1))


def _tc_call(body, ins, n_out, out_dim):
    grid = (NCORE, NP // _BR)
    in_specs = [_rows(a.shape) if a.shape[1] == NP else _whole(a.shape) for a in ins]
    out_shape = [jax.ShapeDtypeStruct((NCORE, NP, out_dim), jnp.float32)] * n_out
    out_specs = [_rows((NCORE, NP, out_dim))] * n_out
    res = pl.pallas_call(
        body, grid=grid, in_specs=in_specs, out_specs=out_specs,
        out_shape=out_shape)(*ins)
    return res


def kernel(x_user, x_item, edge_index_user_rates_item, edge_index_item_rated_by_user,
           Wl1_ui, Wr1_ui, b1_ui, Wl2_ui, Wr2_ui, b2_ui,
           Wl1_iu, Wr1_iu, b1_iu, Wl2_iu, Wr2_iu, b2_iu,
           g1_user, be1_user, g2_user, be2_user, Rw_user, Rb_user, Pw_user, Pb_user,
           g1_item, be1_item, g2_item, be2_item, Rw_item, Rb_item, Pw_item, Pb_item):
    srcs, dsts = _prep_edges(edge_index_user_rates_item, edge_index_item_rated_by_user)
    table1 = jnp.concatenate([x_user, x_item], 0)

    pad = lambda a: jnp.pad(a, ((0, NP - N), (0, 0)))
    x = jnp.stack([pad(x_item), pad(x_user)])
    wl1 = jnp.stack([Wl1_ui, Wl1_iu])
    wr1 = jnp.stack([Wr1_ui, Wr1_iu])
    b1 = jnp.stack([b1_ui, b1_iu])[:, None, :]
    g1 = jnp.stack([g1_item, g1_user])[:, None, :]
    be1 = jnp.stack([be1_item, be1_user])[:, None, :]
    wl2 = jnp.stack([Wl2_iu, Wl2_ui])
    wc = jnp.stack([Wr2_ui + Rw_item, Wr2_iu + Rw_user])
    bc = jnp.stack([b2_ui + Rb_item, b2_iu + Rb_user])[:, None, :]
    g2 = jnp.stack([g2_item, g2_user])[:, None, :]
    be2 = jnp.stack([be2_item, be2_user])[:, None, :]
    pw = jnp.stack([Pw_item, Pw_user])
    pb = jnp.stack([Pb_item, Pb_user])[:, None, :]

    s1, counts = _sc_segsum(table1, srcs, dsts, with_counts=True)
    c = counts[:, :, None]

    y, w = _tc_call(_tc1_body, [s1, c, x, wl1, wr1, b1, g1, be1, wl2, wc, bc],
                    n_out=2, out_dim=O)

    table2 = jnp.concatenate([y[1, :N], y[0, :N]], 0)
    (s2,) = _sc_segsum(table2, srcs, dsts, with_counts=False)

    (out,) = _tc_call(_tc2_body, [s2, c, w, g2, be2, pw, pb], n_out=1, out_dim=O)
    return (out[1, :N], out[0, :N])


# speedup vs baseline: 3.3643x; 1.0012x over previous
import functools

import jax
import jax.numpy as jnp
from jax import lax
from jax.experimental import pallas as pl
from jax.experimental.pallas import tpu as pltpu
from jax.experimental.pallas import tpu_sc as plsc

N = 10000
E = 320000
D = 128
H = 256
O = 128

NP = 10240
CH = 128
NTILE = 16
NCORE = 2
NCHUNK = 160
EW = NCHUNK * CH
EPAD = NTILE * EW
RPT = NP // NTILE


def _prep_edges(ei_ui, ei_iu):
    def prep(ei, off):
        pad = EPAD - E
        src = jnp.concatenate([ei[0] + off, jnp.zeros((pad,), jnp.int32)])
        dst = jnp.concatenate([ei[1], jnp.full((pad,), N, jnp.int32)])
        return src.reshape(NTILE, NCHUNK, CH), dst.reshape(NTILE, NCHUNK, CH)

    su, du = prep(ei_ui, 0)
    si, di = prep(ei_iu, NP)
    return jnp.stack([su, si]), jnp.stack([du, di])


def _sc_segsum(table, srcs, dsts, with_counts):
    mesh = plsc.VectorSubcoreMesh(core_axis_name="c", subcore_axis_name="s")
    out_type = [jax.ShapeDtypeStruct((NCORE, NP, D), jnp.float32)]
    if with_counts:
        out_type.append(jax.ShapeDtypeStruct((NCORE, NP), jnp.float32))
    scratch = [
        pltpu.VMEM((3, CH), jnp.int32),
        pltpu.VMEM((3, CH), jnp.int32),
        pltpu.VMEM((2, CH, D), jnp.float32),
        pltpu.VMEM((CH,), jnp.float32),
        pltpu.VMEM_SHARED((NP, D), jnp.float32),
        pltpu.VMEM_SHARED((NP,), jnp.float32),
        pltpu.SemaphoreType.DMA,
        pltpu.SemaphoreType.DMA,
        pltpu.SemaphoreType.DMA,
        pltpu.SemaphoreType.DMA,
    ]

    @functools.partial(pl.kernel, out_type=out_type, mesh=mesh,
                       scratch_types=scratch)
    def k(table_h, srcs_h, dsts_h, zrows_h, zcnt_h, *rest):
        if with_counts:
            (sums_h, counts_h, src_v, dst_v, rows_v, ones_v, acc, cacc,
             gsem, isem, ssem, csem) = rest
        else:
            (sums_h, src_v, dst_v, rows_v, ones_v, acc, cacc,
             gsem, isem, ssem, csem) = rest
        cid = lax.axis_index("c")
        sid = lax.axis_index("s")
        sl = pl.ds(sid * RPT, RPT)

        def stage_idx(chunk, buf, sync):
            cp = pltpu.sync_copy if sync else (
                lambda s, d: pltpu.async_copy(s, d, isem))
            cp(srcs_h.at[cid, sid, chunk], src_v.at[buf])
            cp(dsts_h.at[cid, sid, chunk], dst_v.at[buf])

        def wait_idx():
            pltpu.make_async_copy(srcs_h.at[cid, sid, 0], src_v.at[0], isem).wait()
            pltpu.make_async_copy(dsts_h.at[cid, sid, 0], dst_v.at[0], isem).wait()

        def gather(chunk_buf, row_buf):
            pltpu.async_copy(table_h.at[src_v.at[chunk_buf]],
                             rows_v.at[row_buf], gsem)

        def wait_gather():
            pltpu.make_async_copy(table_h.at[src_v.at[0]], rows_v.at[0],
                                  gsem).wait()

        def scatter(row_buf, chunk_buf):
            pltpu.async_copy(rows_v.at[row_buf], acc.at[dst_v.at[chunk_buf]],
                             ssem, add=True)
            if with_counts:
                pltpu.async_copy(ones_v, cacc.at[dst_v.at[chunk_buf]],
                                 csem, add=True)

        def wait_scatter():
            pltpu.make_async_copy(rows_v.at[0], acc.at[dst_v.at[0]], ssem).wait()
            if with_counts:
                pltpu.make_async_copy(ones_v, cacc.at[dst_v.at[0]], csem).wait()

        pltpu.sync_copy(zrows_h.at[sl], acc.at[sl])
        if with_counts:
            pltpu.sync_copy(zcnt_h.at[sl], cacc.at[sl])
            for i in range(CH // 16):
                ones_v[pl.ds(i * 16, 16)] = jnp.ones((16,), jnp.float32)

        plsc.subcore_barrier()

        stage_idx(0, 0, sync=True)
        stage_idx(1, 1, sync=False)
        gather(0, 0)

        def body(j, carry):
            @pl.when(j > 0)
            def _():
                wait_scatter()
            wait_idx()
            stage_idx((j + 2) % NCHUNK, (j + 2) % 3, sync=False)
            wait_gather()
            @pl.when(j + 1 < NCHUNK)
            def _():
                gather((j + 1) % 3, (j + 1) % 2)
            scatter(j % 2, j % 3)
            return carry

        lax.fori_loop(0, NCHUNK, body, 0)
        wait_scatter()
        wait_idx()
        plsc.subcore_barrier()

        pltpu.sync_copy(acc.at[sl], sums_h.at[cid, sl])
        if with_counts:
            pltpu.sync_copy(cacc.at[sl], counts_h.at[cid, sl])

    zrows = jnp.zeros((NP, D), jnp.float32)
    zcnt = jnp.zeros((NP,), jnp.float32)
    out = k(table, srcs, dsts, zrows, zcnt)
    return tuple(jax.tree.leaves(out))


_BR = 1280


def _tc1_body(s_ref, c_ref, x_ref, wl1_ref, wr1_ref, b1_ref, g1_ref, be1_ref,
              wl2_ref, wc_ref, bc_ref, y_ref, w_ref):
    cnt = jnp.maximum(c_ref[0], 1.0)
    agg = s_ref[0] / cnt
    h = (jnp.dot(agg, wl1_ref[0], preferred_element_type=jnp.float32)
         + jnp.dot(x_ref[0], wr1_ref[0], preferred_element_type=jnp.float32)
         + b1_ref[0])
    m = jnp.mean(h, axis=-1, keepdims=True)
    v = jnp.mean((h - m) ** 2, axis=-1, keepdims=True)
    x1 = jnp.maximum((h - m) * lax.rsqrt(v + 1e-5) * g1_ref[0] + be1_ref[0], 0.0)
    y_ref[0] = jnp.dot(x1, wl2_ref[0], preferred_element_type=jnp.float32)
    w_ref[0] = (jnp.dot(x1, wc_ref[0], preferred_element_type=jnp.float32)
                + bc_ref[0])


def _tc2_body(s_ref, c_ref, w_ref, g2_ref, be2_ref, pw_ref, pb_ref, o_ref):
    cnt = jnp.maximum(c_ref[0], 1.0)
    t = s_ref[0] / cnt + w_ref[0]
    m = jnp.mean(t, axis=-1, keepdims=True)
    v = jnp.mean((t - m) ** 2, axis=-1, keepdims=True)
    x2 = (t - m) * lax.rsqrt(v + 1e-5) * g2_ref[0] + be2_ref[0]
    o_ref[0] = jnp.dot(x2, pw_ref[0], preferred_element_type=jnp.float32) + pb_ref[0]


def _rows(shape):
    return pl.BlockSpec((1, _BR) + shape[2:], lambda t, i: (t, i) + (0,) * (len(shape) - 2))


def _whole(shape):
    return pl.BlockSpec((1,) + shape[1:], lambda t, i: (t,) + (0,) * (len(shape) - 1))


def _tc_call(body, ins, n_out, out_dim):
    grid = (NCORE, NP // _BR)
    in_specs = [_rows(a.shape) if a.shape[1] == NP else _whole(a.shape) for a in ins]
    out_shape = [jax.ShapeDtypeStruct((NCORE, NP, out_dim), jnp.float32)] * n_out
    out_specs = [_rows((NCORE, NP, out_dim))] * n_out
    res = pl.pallas_call(
        body, grid=grid, in_specs=in_specs, out_specs=out_specs,
        out_shape=out_shape)(*ins)
    return res


def kernel(x_user, x_item, edge_index_user_rates_item, edge_index_item_rated_by_user,
           Wl1_ui, Wr1_ui, b1_ui, Wl2_ui, Wr2_ui, b2_ui,
           Wl1_iu, Wr1_iu, b1_iu, Wl2_iu, Wr2_iu, b2_iu,
           g1_user, be1_user, g2_user, be2_user, Rw_user, Rb_user, Pw_user, Pb_user,
           g1_item, be1_item, g2_item, be2_item, Rw_item, Rb_item, Pw_item, Pb_item):
    srcs, dsts = _prep_edges(edge_index_user_rates_item, edge_index_item_rated_by_user)
    pad = lambda a: jnp.pad(a, ((0, NP - N), (0, 0)))
    table1 = jnp.concatenate([pad(x_user), pad(x_item)], 0)
    x = jnp.stack([pad(x_item), pad(x_user)])
    wl1 = jnp.stack([Wl1_ui, Wl1_iu])
    wr1 = jnp.stack([Wr1_ui, Wr1_iu])
    b1 = jnp.stack([b1_ui, b1_iu])[:, None, :]
    g1 = jnp.stack([g1_item, g1_user])[:, None, :]
    be1 = jnp.stack([be1_item, be1_user])[:, None, :]
    wl2 = jnp.stack([Wl2_iu, Wl2_ui])
    wc = jnp.stack([Wr2_ui + Rw_item, Wr2_iu + Rw_user])
    bc = jnp.stack([b2_ui + Rb_item, b2_iu + Rb_user])[:, None, :]
    g2 = jnp.stack([g2_item, g2_user])[:, None, :]
    be2 = jnp.stack([be2_item, be2_user])[:, None, :]
    pw = jnp.stack([Pw_item, Pw_user])
    pb = jnp.stack([Pb_item, Pb_user])[:, None, :]

    s1, counts = _sc_segsum(table1, srcs, dsts, with_counts=True)
    c = counts[:, :, None]

    y, w = _tc_call(_tc1_body, [s1, c, x, wl1, wr1, b1, g1, be1, wl2, wc, bc],
                    n_out=2, out_dim=O)

    table2 = jnp.concatenate([y[1], y[0]], 0)
    (s2,) = _sc_segsum(table2, srcs, dsts, with_counts=False)

    (out,) = _tc_call(_tc2_body, [s2, c, w, g2, be2, pw, pb], n_out=1, out_dim=O)
    return (out[1, :N], out[0, :N])
